# baseline (device time: 118938 ns/iter reference)
import jax
import jax.numpy as jnp
from jax import lax
from jax.experimental import pallas as pl
from jax.experimental.pallas import tpu as pltpu

N_DEV = 4


def kernel(x, w_mat, scale_x, scale_w):
    m_per, k = x.shape
    _, n_per = w_mat.shape
    m_tot = N_DEV * m_per
    mh = m_per // 2
    mq = m_per // 4
    kq = k // 4

    x8 = x.astype(jnp.float8_e5m2)

    def body(scale_x_ref, scale_w_ref, x_ref, w_hbm, out_hbm,
             xg_ref, w8_ref, wst_ref, ost_ref,
             send_sems, recv_sems, wdma_sems, odma_sems):
        my = lax.axis_index("i")
        left = (my - 1) % N_DEV
        right = (my + 1) % N_DEV
        s = scale_x_ref[0] * scale_w_ref[0]

        FROM_L, FROM_R, DIAG = 0, 1, 2


        def rdma(i, src, dst, dev):
            return pltpu.make_async_remote_copy(
                src_ref=src, dst_ref=dst,
                send_sem=send_sems.at[i], recv_sem=recv_sems.at[i],
                device_id=(dev,), device_id_type=pl.DeviceIdType.MESH,
            )

        hop1 = [
            rdma(0, x_ref.at[pl.ds(0, mh)], xg_ref.at[FROM_L, pl.ds(0, mh)], right),
            rdma(1, x_ref.at[pl.ds(mh, mh)], xg_ref.at[FROM_L, pl.ds(mh, mh)], right),
            rdma(2, x_ref.at[pl.ds(0, mh)], xg_ref.at[FROM_R, pl.ds(0, mh)], left),
            rdma(3, x_ref.at[pl.ds(mh, mh)], xg_ref.at[FROM_R, pl.ds(mh, mh)], left),
        ]
        for r in hop1:
            r.start()

        fwd = [
            rdma(4, xg_ref.at[FROM_L, pl.ds(0, mq)], xg_ref.at[DIAG, pl.ds(0, mq)], right),
            rdma(5, xg_ref.at[FROM_L, pl.ds(mq, mq)], xg_ref.at[DIAG, pl.ds(mq, mq)], right),
            rdma(6, xg_ref.at[FROM_R, pl.ds(mh, mq)], xg_ref.at[DIAG, pl.ds(mh, mq)], left),
            rdma(7, xg_ref.at[FROM_R, pl.ds(mh + mq, mq)], xg_ref.at[DIAG, pl.ds(mh + mq, mq)], left),
        ]

        wcp = [
            pltpu.make_async_copy(
                w_hbm.at[pl.ds(q * kq, kq)], wst_ref.at[q % 2],
                wdma_sems.at[q % 2])
            for q in range(4)
        ]
        wcp[0].start()
        wcp[1].start()
        for q in range(4):
            wcp[q].wait()
            w8_ref[pl.ds(q * kq, kq), :] = wst_ref[q % 2].astype(
                jnp.float8_e5m2)
            if q + 2 < 4:
                wcp[q + 2].start()

        pending = [None, None]
        next_slot = [0]

        def gemm(x_chunk, out_row, rows):
            slot = next_slot[0]
            next_slot[0] ^= 1
            if pending[slot] is not None:
                pending[slot].wait()
            acc = jnp.dot(x_chunk, w8_ref[...],
                          preferred_element_type=jnp.float32)
            ost_ref[slot, pl.ds(0, rows)] = jnp.maximum(acc * s, 0.0)
            cp = pltpu.make_async_copy(
                ost_ref.at[slot, pl.ds(0, rows)],
                out_hbm.at[pl.ds(out_row, rows)],
                odma_sems.at[slot],
            )
            cp.start()
            pending[slot] = cp

        gemm(x_ref[pl.ds(0, mh)], my * m_per, mh)
        gemm(x_ref[pl.ds(mh, mh)], my * m_per + mh, mh)

        hop1[0].wait_recv()
        fwd[0].start()
        fwd[1].start()
        gemm(xg_ref[FROM_L, pl.ds(0, mh)], left * m_per, mh)
        hop1[2].wait_recv()
        gemm(xg_ref[FROM_R, pl.ds(0, mh)], right * m_per, mh)
        hop1[1].wait_recv()
        gemm(xg_ref[FROM_L, pl.ds(mh, mh)], left * m_per + mh, mh)
        hop1[3].wait_recv()
        fwd[2].start()
        fwd[3].start()
        gemm(xg_ref[FROM_R, pl.ds(mh, mh)], right * m_per + mh, mh)

        diag_row = ((my + 2) % N_DEV) * m_per
        fwd[0].wait_recv()
        gemm(xg_ref[DIAG, pl.ds(0, mq)], diag_row, mq)
        fwd[2].wait_recv()
        gemm(xg_ref[DIAG, pl.ds(mh, mq)], diag_row + mh, mq)
        fwd[1].wait_recv()
        gemm(xg_ref[DIAG, pl.ds(mq, mq)], diag_row + mq, mq)
        fwd[3].wait_recv()
        gemm(xg_ref[DIAG, pl.ds(mh + mq, mq)], diag_row + mh + mq, mq)

        for p in pending:
            p.wait()
        for r in hop1 + fwd:
            r.wait_send()

    return pl.pallas_call(
        body,
        out_shape=jax.ShapeDtypeStruct((m_tot, n_per), jnp.float32),
        in_specs=[
            pl.BlockSpec(memory_space=pltpu.SMEM),
            pl.BlockSpec(memory_space=pltpu.SMEM),
            pl.BlockSpec(memory_space=pltpu.VMEM),
            pl.BlockSpec(memory_space=pl.ANY),
        ],
        out_specs=pl.BlockSpec(memory_space=pl.ANY),
        scratch_shapes=[
            pltpu.VMEM((3, m_per, k), jnp.float8_e5m2),
            pltpu.VMEM((k, n_per), jnp.float8_e5m2),
            pltpu.VMEM((2, kq, n_per), jnp.float32),
            pltpu.VMEM((2, mh, n_per), jnp.float32),
            pltpu.SemaphoreType.DMA((8,)),
            pltpu.SemaphoreType.DMA((8,)),
            pltpu.SemaphoreType.DMA((2,)),
            pltpu.SemaphoreType.DMA((2,)),
        ],
        compiler_params=pltpu.CompilerParams(
            vmem_limit_bytes=100 * 1024 * 1024,
        ),
    )(scale_x, scale_w, x8, w_mat)


# device time: 109182 ns/iter; 1.0894x vs baseline; 1.0894x over previous
import jax
import jax.numpy as jnp
from jax import lax
from jax.experimental import pallas as pl
from jax.experimental.pallas import tpu as pltpu

N_DEV = 4


def kernel(x, w_mat, scale_x, scale_w):
    m_per, k = x.shape
    _, n_per = w_mat.shape
    m_tot = N_DEV * m_per
    mh = m_per // 2
    mq = m_per // 4
    kp = k // 8

    def body(scale_x_ref, scale_w_ref, x_hbm, w_hbm, out_hbm,
             xg_ref, w8_ref, wst_ref, xst_ref, ost_ref,
             send_sems, recv_sems, wdma_sems, xdma_sems, odma_sems):
        my = lax.axis_index("i")
        left = (my - 1) % N_DEV
        right = (my + 1) % N_DEV
        s = scale_x_ref[0] * scale_w_ref[0]

        FROM_L, FROM_R, DIAG, OWN = 0, 1, 2, 3

        barrier = pltpu.get_barrier_semaphore()
        for nbr in (left, right):
            pl.semaphore_signal(
                barrier, inc=1,
                device_id=(nbr,), device_id_type=pl.DeviceIdType.MESH,
            )
        pl.semaphore_wait(barrier, 2)

        def rdma(i, src, dst, dev):
            return pltpu.make_async_remote_copy(
                src_ref=src, dst_ref=dst,
                send_sem=send_sems.at[i], recv_sem=recv_sems.at[i],
                device_id=(dev,), device_id_type=pl.DeviceIdType.MESH,
            )

        hop1 = [
            rdma(0, xg_ref.at[OWN, pl.ds(0, mh)], xg_ref.at[FROM_L, pl.ds(0, mh)], right),
            rdma(1, xg_ref.at[OWN, pl.ds(mh, mh)], xg_ref.at[FROM_L, pl.ds(mh, mh)], right),
            rdma(2, xg_ref.at[OWN, pl.ds(0, mh)], xg_ref.at[FROM_R, pl.ds(0, mh)], left),
            rdma(3, xg_ref.at[OWN, pl.ds(mh, mh)], xg_ref.at[FROM_R, pl.ds(mh, mh)], left),
        ]

        fwd = [
            rdma(4, xg_ref.at[FROM_L, pl.ds(0, mq)], xg_ref.at[DIAG, pl.ds(0, mq)], right),
            rdma(5, xg_ref.at[FROM_L, pl.ds(mq, mq)], xg_ref.at[DIAG, pl.ds(mq, mq)], right),
            rdma(6, xg_ref.at[FROM_R, pl.ds(mh, mq)], xg_ref.at[DIAG, pl.ds(mh, mq)], left),
            rdma(7, xg_ref.at[FROM_R, pl.ds(mh + mq, mq)], xg_ref.at[DIAG, pl.ds(mh + mq, mq)], left),
        ]

        xcp = [
            pltpu.make_async_copy(
                x_hbm.at[pl.ds(q * mq, mq)], xst_ref.at[q % 2],
                xdma_sems.at[q % 2])
            for q in range(4)
        ]
        xcp[0].start()
        xcp[1].start()
        for q in range(4):
            xcp[q].wait()
            xg_ref[OWN, pl.ds(q * mq, mq), :] = xst_ref[q % 2].astype(
                jnp.float8_e5m2)
            if q + 2 < 4:
                xcp[q + 2].start()
            if q == 1:
                hop1[0].start()
                hop1[2].start()
            if q == 3:
                hop1[1].start()
                hop1[3].start()

        wcp = [
            pltpu.make_async_copy(
                w_hbm.at[pl.ds(p * kp, kp)], wst_ref.at[p % 2],
                wdma_sems.at[p % 2])
            for p in range(8)
        ]
        wcp[0].start()
        wcp[1].start()
        for p in range(8):
            wcp[p].wait()
            w8_ref[pl.ds(p * kp, kp), :] = wst_ref[p % 2].astype(
                jnp.float8_e5m2)
            if p + 2 < 8:
                wcp[p + 2].start()

        pending = [None, None]
        next_slot = [0]

        def gemm(x_chunk, out_row, rows):
            slot = next_slot[0]
            next_slot[0] ^= 1
            if pending[slot] is not None:
                pending[slot].wait()
            acc = jnp.dot(x_chunk, w8_ref[...],
                          preferred_element_type=jnp.float32)
            ost_ref[slot, pl.ds(0, rows)] = jnp.maximum(acc * s, 0.0)
            cp = pltpu.make_async_copy(
                ost_ref.at[slot, pl.ds(0, rows)],
                out_hbm.at[pl.ds(out_row, rows)],
                odma_sems.at[slot],
            )
            cp.start()
            pending[slot] = cp

        gemm(xg_ref[OWN, pl.ds(0, mh)], my * m_per, mh)
        gemm(xg_ref[OWN, pl.ds(mh, mh)], my * m_per + mh, mh)

        hop1[0].wait_recv()
        fwd[0].start()
        fwd[1].start()
        gemm(xg_ref[FROM_L, pl.ds(0, mh)], left * m_per, mh)
        hop1[2].wait_recv()
        gemm(xg_ref[FROM_R, pl.ds(0, mh)], right * m_per, mh)
        hop1[1].wait_recv()
        gemm(xg_ref[FROM_L, pl.ds(mh, mh)], left * m_per + mh, mh)
        hop1[3].wait_recv()
        fwd[2].start()
        fwd[3].start()
        gemm(xg_ref[FROM_R, pl.ds(mh, mh)], right * m_per + mh, mh)

        diag_row = ((my + 2) % N_DEV) * m_per
        fwd[0].wait_recv()
        gemm(xg_ref[DIAG, pl.ds(0, mq)], diag_row, mq)
        fwd[2].wait_recv()
        gemm(xg_ref[DIAG, pl.ds(mh, mq)], diag_row + mh, mq)
        fwd[1].wait_recv()
        gemm(xg_ref[DIAG, pl.ds(mq, mq)], diag_row + mq, mq)
        fwd[3].wait_recv()
        gemm(xg_ref[DIAG, pl.ds(mh + mq, mq)], diag_row + mh + mq, mq)

        for p in pending:
            p.wait()
        for r in hop1 + fwd:
            r.wait_send()

    return pl.pallas_call(
        body,
        out_shape=jax.ShapeDtypeStruct((m_tot, n_per), jnp.float32),
        in_specs=[
            pl.BlockSpec(memory_space=pltpu.SMEM),
            pl.BlockSpec(memory_space=pltpu.SMEM),
            pl.BlockSpec(memory_space=pl.ANY),
            pl.BlockSpec(memory_space=pl.ANY),
        ],
        out_specs=pl.BlockSpec(memory_space=pl.ANY),
        scratch_shapes=[
            pltpu.VMEM((4, m_per, k), jnp.float8_e5m2),
            pltpu.VMEM((k, n_per), jnp.float8_e5m2),
            pltpu.VMEM((2, kp, n_per), jnp.float32),
            pltpu.VMEM((2, mq, k), jnp.float32),
            pltpu.VMEM((2, mh, n_per), jnp.float32),
            pltpu.SemaphoreType.DMA((8,)),
            pltpu.SemaphoreType.DMA((8,)),
            pltpu.SemaphoreType.DMA((2,)),
            pltpu.SemaphoreType.DMA((2,)),
            pltpu.SemaphoreType.DMA((2,)),
        ],
        compiler_params=pltpu.CompilerParams(
            collective_id=0,
            vmem_limit_bytes=100 * 1024 * 1024,
        ),
    )(scale_x, scale_w, x, w_mat)


# device time: 105153 ns/iter; 1.1311x vs baseline; 1.0383x over previous
import jax
import jax.numpy as jnp
from jax import lax
from jax.experimental import pallas as pl
from jax.experimental.pallas import tpu as pltpu

N_DEV = 4


def kernel(x, w_mat, scale_x, scale_w):
    m_per, k = x.shape
    _, n_per = w_mat.shape
    m_tot = N_DEV * m_per
    mh = m_per // 2
    mq = m_per // 4
    kp = k // 8

    def body(scale_x_ref, scale_w_ref, x_hbm, w_hbm, out_hbm,
             xg_ref, w8_ref, wst_ref, xst_ref, ost_ref,
             send_sems, recv_sems, wdma_sems, xdma_sems, odma_sems):
        my = lax.axis_index("i")
        left = (my - 1) % N_DEV
        right = (my + 1) % N_DEV
        s = scale_x_ref[0] * scale_w_ref[0]

        FROM_L, FROM_R, DIAG, OWN = 0, 1, 2, 3

        def rdma(i, src, dst, dev):
            return pltpu.make_async_remote_copy(
                src_ref=src, dst_ref=dst,
                send_sem=send_sems.at[i], recv_sem=recv_sems.at[i],
                device_id=(dev,), device_id_type=pl.DeviceIdType.MESH,
            )

        hop1 = [
            rdma(0, xg_ref.at[OWN, pl.ds(0, mh)], xg_ref.at[FROM_L, pl.ds(0, mh)], right),
            rdma(1, xg_ref.at[OWN, pl.ds(mh, mh)], xg_ref.at[FROM_L, pl.ds(mh, mh)], right),
            rdma(2, xg_ref.at[OWN, pl.ds(0, mh)], xg_ref.at[FROM_R, pl.ds(0, mh)], left),
            rdma(3, xg_ref.at[OWN, pl.ds(mh, mh)], xg_ref.at[FROM_R, pl.ds(mh, mh)], left),
        ]

        mo = mq // 2
        fwd = [
            rdma(4, xg_ref.at[FROM_L, pl.ds(0, mq)], xg_ref.at[DIAG, pl.ds(0, mq)], right),
            rdma(5, xg_ref.at[FROM_L, pl.ds(mq, mo)], xg_ref.at[DIAG, pl.ds(mq, mo)], right),
            rdma(6, xg_ref.at[FROM_L, pl.ds(mq + mo, mo)], xg_ref.at[DIAG, pl.ds(mq + mo, mo)], right),
            rdma(7, xg_ref.at[FROM_R, pl.ds(mh, mq)], xg_ref.at[DIAG, pl.ds(mh, mq)], left),
            rdma(8, xg_ref.at[FROM_R, pl.ds(mh + mq, mo)], xg_ref.at[DIAG, pl.ds(mh + mq, mo)], left),
            rdma(9, xg_ref.at[FROM_R, pl.ds(mh + mq + mo, mo)], xg_ref.at[DIAG, pl.ds(mh + mq + mo, mo)], left),
        ]

        xcp = [
            pltpu.make_async_copy(
                x_hbm.at[pl.ds(q * mq, mq)], xst_ref.at[q % 2],
                xdma_sems.at[q % 2])
            for q in range(4)
        ]
        xcp[0].start()
        xcp[1].start()

        barrier = pltpu.get_barrier_semaphore()
        for nbr in (left, right):
            pl.semaphore_signal(
                barrier, inc=1,
                device_id=(nbr,), device_id_type=pl.DeviceIdType.MESH,
            )
        pl.semaphore_wait(barrier, 2)

        for q in range(4):
            xcp[q].wait()
            xg_ref[OWN, pl.ds(q * mq, mq), :] = xst_ref[q % 2].astype(
                jnp.float8_e5m2)
            if q + 2 < 4:
                xcp[q + 2].start()
            if q == 1:
                hop1[0].start()
                hop1[2].start()
            if q == 3:
                hop1[1].start()
                hop1[3].start()

        wcp = [
            pltpu.make_async_copy(
                w_hbm.at[pl.ds(p * kp, kp)], wst_ref.at[p % 2],
                wdma_sems.at[p % 2])
            for p in range(8)
        ]
        wcp[0].start()
        wcp[1].start()
        for p in range(8):
            wcp[p].wait()
            w8_ref[pl.ds(p * kp, kp), :] = wst_ref[p % 2].astype(
                jnp.float8_e5m2)
            if p + 2 < 8:
                wcp[p + 2].start()

        pending = [None, None]
        next_slot = [0]

        def gemm(x_chunk, out_row, rows):
            slot = next_slot[0]
            next_slot[0] ^= 1
            if pending[slot] is not None:
                pending[slot].wait()
            acc = jnp.dot(x_chunk, w8_ref[...],
                          preferred_element_type=jnp.float32)
            ost_ref[slot, pl.ds(0, rows)] = jnp.maximum(acc * s, 0.0)
            cp = pltpu.make_async_copy(
                ost_ref.at[slot, pl.ds(0, rows)],
                out_hbm.at[pl.ds(out_row, rows)],
                odma_sems.at[slot],
            )
            cp.start()
            pending[slot] = cp

        gemm(xg_ref[OWN, pl.ds(0, mh)], my * m_per, mh)
        gemm(xg_ref[OWN, pl.ds(mh, mh)], my * m_per + mh, mh)

        hop1[0].wait_recv()
        for i in (0, 1, 2):
            fwd[i].start()
        gemm(xg_ref[FROM_L, pl.ds(0, mh)], left * m_per, mh)
        hop1[2].wait_recv()
        gemm(xg_ref[FROM_R, pl.ds(0, mh)], right * m_per, mh)
        hop1[1].wait_recv()
        gemm(xg_ref[FROM_L, pl.ds(mh, mh)], left * m_per + mh, mh)
        hop1[3].wait_recv()
        for i in (3, 4, 5):
            fwd[i].start()
        gemm(xg_ref[FROM_R, pl.ds(mh, mh)], right * m_per + mh, mh)

        diag_row = ((my + 2) % N_DEV) * m_per
        fwd[0].wait_recv()
        gemm(xg_ref[DIAG, pl.ds(0, mq)], diag_row, mq)
        fwd[3].wait_recv()
        gemm(xg_ref[DIAG, pl.ds(mh, mq)], diag_row + mh, mq)
        fwd[1].wait_recv()
        gemm(xg_ref[DIAG, pl.ds(mq, mo)], diag_row + mq, mo)
        fwd[4].wait_recv()
        gemm(xg_ref[DIAG, pl.ds(mh + mq, mo)], diag_row + mh + mq, mo)
        fwd[2].wait_recv()
        gemm(xg_ref[DIAG, pl.ds(mq + mo, mo)], diag_row + mq + mo, mo)
        fwd[5].wait_recv()
        gemm(xg_ref[DIAG, pl.ds(mh + mq + mo, mo)], diag_row + mh + mq + mo, mo)

        for p in pending:
            p.wait()
        for r in hop1 + fwd:
            r.wait_send()

    return pl.pallas_call(
        body,
        out_shape=jax.ShapeDtypeStruct((m_tot, n_per), jnp.float32),
        in_specs=[
            pl.BlockSpec(memory_space=pltpu.SMEM),
            pl.BlockSpec(memory_space=pltpu.SMEM),
            pl.BlockSpec(memory_space=pl.ANY),
            pl.BlockSpec(memory_space=pl.ANY),
        ],
        out_specs=pl.BlockSpec(memory_space=pl.ANY),
        scratch_shapes=[
            pltpu.VMEM((4, m_per, k), jnp.float8_e5m2),
            pltpu.VMEM((k, n_per), jnp.float8_e5m2),
            pltpu.VMEM((2, kp, n_per), jnp.float32),
            pltpu.VMEM((2, mq, k), jnp.float32),
            pltpu.VMEM((2, mh, n_per), jnp.float32),
            pltpu.SemaphoreType.DMA((10,)),
            pltpu.SemaphoreType.DMA((10,)),
            pltpu.SemaphoreType.DMA((2,)),
            pltpu.SemaphoreType.DMA((2,)),
            pltpu.SemaphoreType.DMA((2,)),
        ],
        compiler_params=pltpu.CompilerParams(
            collective_id=0,
            vmem_limit_bytes=100 * 1024 * 1024,
        ),
    )(scale_x, scale_w, x, w_mat)


# device time: 104080 ns/iter; 1.1428x vs baseline; 1.0103x over previous
import jax
import jax.numpy as jnp
from jax import lax
from jax.experimental import pallas as pl
from jax.experimental.pallas import tpu as pltpu

N_DEV = 4


def kernel(x, w_mat, scale_x, scale_w):
    m_per, k = x.shape
    _, n_per = w_mat.shape
    m_tot = N_DEV * m_per
    mh = m_per // 2
    mq = m_per // 4
    kp = k // 8

    def body(scale_x_ref, scale_w_ref, x_hbm, w_hbm, out_hbm,
             xg_ref, w8_ref, wst_ref, xst_ref, ost_ref,
             send_sems, recv_sems, wdma_sems, xdma_sems, odma_sems):
        my = lax.axis_index("i")
        left = (my - 1) % N_DEV
        right = (my + 1) % N_DEV
        s = scale_x_ref[0] * scale_w_ref[0]

        FROM_L, FROM_R, DIAG, OWN = 0, 1, 2, 3

        def rdma(i, src, dst, dev):
            return pltpu.make_async_remote_copy(
                src_ref=src, dst_ref=dst,
                send_sem=send_sems.at[i], recv_sem=recv_sems.at[i],
                device_id=(dev,), device_id_type=pl.DeviceIdType.MESH,
            )

        hop1 = [
            rdma(q, xg_ref.at[OWN, pl.ds(q * mq, mq)],
                 xg_ref.at[FROM_L, pl.ds(q * mq, mq)], right)
            for q in range(4)
        ] + [
            rdma(4 + q, xg_ref.at[OWN, pl.ds(q * mq, mq)],
                 xg_ref.at[FROM_R, pl.ds(q * mq, mq)], left)
            for q in range(4)
        ]

        mo = mq // 2
        fwd = [
            rdma(8, xg_ref.at[FROM_L, pl.ds(0, mq)], xg_ref.at[DIAG, pl.ds(0, mq)], right),
            rdma(9, xg_ref.at[FROM_L, pl.ds(mq, mo)], xg_ref.at[DIAG, pl.ds(mq, mo)], right),
            rdma(10, xg_ref.at[FROM_L, pl.ds(mq + mo, mo)], xg_ref.at[DIAG, pl.ds(mq + mo, mo)], right),
            rdma(11, xg_ref.at[FROM_R, pl.ds(mh, mq)], xg_ref.at[DIAG, pl.ds(mh, mq)], left),
            rdma(12, xg_ref.at[FROM_R, pl.ds(mh + mq, mo)], xg_ref.at[DIAG, pl.ds(mh + mq, mo)], left),
            rdma(13, xg_ref.at[FROM_R, pl.ds(mh + mq + mo, mo)], xg_ref.at[DIAG, pl.ds(mh + mq + mo, mo)], left),
        ]

        xcp = [
            pltpu.make_async_copy(
                x_hbm.at[pl.ds(q * mq, mq)], xst_ref.at[q % 2],
                xdma_sems.at[q % 2])
            for q in range(4)
        ]
        xcp[0].start()
        xcp[1].start()

        barrier = pltpu.get_barrier_semaphore()
        for nbr in (left, right):
            pl.semaphore_signal(
                barrier, inc=1,
                device_id=(nbr,), device_id_type=pl.DeviceIdType.MESH,
            )
        pl.semaphore_wait(barrier, 2)

        for q in range(4):
            xcp[q].wait()
            xg_ref[OWN, pl.ds(q * mq, mq), :] = xst_ref[q % 2].astype(
                jnp.float8_e5m2)
            if q + 2 < 4:
                xcp[q + 2].start()
            hop1[q].start()
            hop1[4 + q].start()

        wcp = [
            pltpu.make_async_copy(
                w_hbm.at[pl.ds(p * kp, kp)], wst_ref.at[p % 2],
                wdma_sems.at[p % 2])
            for p in range(8)
        ]
        wcp[0].start()
        wcp[1].start()
        for p in range(8):
            wcp[p].wait()
            w8_ref[pl.ds(p * kp, kp), :] = wst_ref[p % 2].astype(
                jnp.float8_e5m2)
            if p + 2 < 8:
                wcp[p + 2].start()

        pending = [None, None]
        next_slot = [0]

        def gemm(x_chunk, out_row, rows):
            slot = next_slot[0]
            next_slot[0] ^= 1
            if pending[slot] is not None:
                pending[slot].wait()
            acc = jnp.dot(x_chunk, w8_ref[...],
                          preferred_element_type=jnp.float32)
            ost_ref[slot, pl.ds(0, rows)] = jnp.maximum(acc * s, 0.0)
            cp = pltpu.make_async_copy(
                ost_ref.at[slot, pl.ds(0, rows)],
                out_hbm.at[pl.ds(out_row, rows)],
                odma_sems.at[slot],
            )
            cp.start()
            pending[slot] = cp

        gemm(xg_ref[OWN, pl.ds(0, mh)], my * m_per, mh)
        gemm(xg_ref[OWN, pl.ds(mh, mh)], my * m_per + mh, mh)

        hop1[0].wait_recv()
        hop1[1].wait_recv()
        for i in (0, 1, 2):
            fwd[i].start()
        gemm(xg_ref[FROM_L, pl.ds(0, mh)], left * m_per, mh)
        hop1[4].wait_recv()
        hop1[5].wait_recv()
        gemm(xg_ref[FROM_R, pl.ds(0, mh)], right * m_per, mh)
        hop1[2].wait_recv()
        hop1[3].wait_recv()
        gemm(xg_ref[FROM_L, pl.ds(mh, mh)], left * m_per + mh, mh)
        hop1[6].wait_recv()
        hop1[7].wait_recv()
        for i in (3, 4, 5):
            fwd[i].start()
        gemm(xg_ref[FROM_R, pl.ds(mh, mh)], right * m_per + mh, mh)

        diag_row = ((my + 2) % N_DEV) * m_per
        fwd[0].wait_recv()
        gemm(xg_ref[DIAG, pl.ds(0, mq)], diag_row, mq)
        fwd[3].wait_recv()
        gemm(xg_ref[DIAG, pl.ds(mh, mq)], diag_row + mh, mq)
        fwd[1].wait_recv()
        gemm(xg_ref[DIAG, pl.ds(mq, mo)], diag_row + mq, mo)
        fwd[4].wait_recv()
        gemm(xg_ref[DIAG, pl.ds(mh + mq, mo)], diag_row + mh + mq, mo)
        fwd[2].wait_recv()
        gemm(xg_ref[DIAG, pl.ds(mq + mo, mo)], diag_row + mq + mo, mo)
        fwd[5].wait_recv()
        gemm(xg_ref[DIAG, pl.ds(mh + mq + mo, mo)], diag_row + mh + mq + mo, mo)

        for p in pending:
            p.wait()
        for r in hop1 + fwd:
            r.wait_send()

    return pl.pallas_call(
        body,
        out_shape=jax.ShapeDtypeStruct((m_tot, n_per), jnp.float32),
        in_specs=[
            pl.BlockSpec(memory_space=pltpu.SMEM),
            pl.BlockSpec(memory_space=pltpu.SMEM),
            pl.BlockSpec(memory_space=pl.ANY),
            pl.BlockSpec(memory_space=pl.ANY),
        ],
        out_specs=pl.BlockSpec(memory_space=pl.ANY),
        scratch_shapes=[
            pltpu.VMEM((4, m_per, k), jnp.float8_e5m2),
            pltpu.VMEM((k, n_per), jnp.float8_e5m2),
            pltpu.VMEM((2, kp, n_per), jnp.float32),
            pltpu.VMEM((2, mq, k), jnp.float32),
            pltpu.VMEM((2, mh, n_per), jnp.float32),
            pltpu.SemaphoreType.DMA((14,)),
            pltpu.SemaphoreType.DMA((14,)),
            pltpu.SemaphoreType.DMA((2,)),
            pltpu.SemaphoreType.DMA((2,)),
            pltpu.SemaphoreType.DMA((2,)),
        ],
        compiler_params=pltpu.CompilerParams(
            collective_id=0,
            vmem_limit_bytes=100 * 1024 * 1024,
        ),
    )(scale_x, scale_w, x, w_mat)
